# Initial kernel scaffold; baseline (speedup 1.0000x reference)
import jax
import jax.numpy as jnp
from jax.experimental import pallas as pl


def kernel(query_embeddings, context_embeddings, context_mask, Wq, bq, Wc, bc, W1, b1, W2, b2):
    raise NotImplementedError("write your pallas kernel here")


# speedup vs baseline: 1.8448x; 1.8448x over previous
import functools

import jax
import jax.numpy as jnp
from jax import lax
from jax.experimental import pallas as pl
from jax.experimental.pallas import tpu as pltpu
from jax.experimental.pallas import tpu_sc as plsc

_BASE_BUDGET = 512.0
_ALPHA = 0.5
_TL = 512


def _dense_body(qe_ref, ce_ref, mask_ref, maskt_ref, wq_ref, bq_ref, wc_ref,
                bc_ref, w1_ref, b1_ref, w2_ref, b2_ref,
                rel_ref, keys_ref, bud_ref):

    ce = ce_ref[0]
    raw = lax.dot_general(ce, wc_ref[...], (((1,), (1,)), ((), ())),
                          preferred_element_type=jnp.float32)
    raw = raw + bc_ref[...]

    qe = qe_ref[0]
    qr = lax.dot_general(qe, wq_ref[...], (((1,), (1,)), ((), ())),
                         preferred_element_type=jnp.float32) + bq_ref[...]
    qn = jnp.sqrt(jnp.sum(qr * qr, axis=1, keepdims=True))
    qr = qr / jnp.maximum(qn, 1e-12)
    qp = jnp.mean(qr, axis=0, keepdims=True)
    qpn = jnp.sqrt(jnp.sum(qp * qp, axis=1, keepdims=True))
    qp = qp / jnp.maximum(qpn, 1e-12)

    n = jnp.sqrt(jnp.sum(raw * raw, axis=1, keepdims=True))
    chat = raw / jnp.maximum(n, 1e-12)
    rel = lax.dot_general(qp, chat, (((1,), (1,)), ((), ())),
                          preferred_element_type=jnp.float32)

    mrow = mask_ref[0]
    mt = maskt_ref[0]
    rel = jnp.where(mt > 0.0, rel, -jnp.inf)
    rel_ref[0] = rel

    u = lax.bitcast_convert_type(rel, jnp.uint32)
    key = u ^ jnp.where(u >> 31 == 1, jnp.uint32(0xFFFFFFFF), jnp.uint32(0x80000000))
    keys_ref[0] = key

    pooled = jnp.mean(qe, axis=0, keepdims=True)
    hidden = lax.dot_general(pooled, w1_ref[...], (((1,), (1,)), ((), ())),
                             preferred_element_type=jnp.float32) + b1_ref[...]
    hidden = jnp.maximum(hidden, 0.0)
    logit = jnp.sum(hidden * w2_ref[...]) + b2_ref[0, 0]
    cx = 1.0 / (1.0 + jnp.exp(-logit))
    budf = jnp.round(_BASE_BUDGET * (1.0 + _ALPHA * cx))
    msum = jnp.sum(mrow)
    budm = jnp.minimum(budf, msum)
    bud_ref[0] = jnp.full((1, 128), budm, jnp.float32).astype(jnp.int32)


def _dense(qe, ce, mask, Wq, bq, Wc, bc, W1, b1, W2, b2):
    Bn, Qn, Dn = qe.shape
    Ln = ce.shape[1]
    Rn = Wq.shape[0]
    Hn = W1.shape[0]
    grid = (Bn, Ln // _TL)
    mask3 = mask.reshape(Bn, 1, Ln)
    out_shape = (
        jax.ShapeDtypeStruct((Bn, 1, Ln), jnp.float32),
        jax.ShapeDtypeStruct((Bn, 1, Ln), jnp.uint32),
        jax.ShapeDtypeStruct((Bn, 1, 128), jnp.int32),
    )
    rel, keys, bud = pl.pallas_call(
        _dense_body,
        grid=grid,
        in_specs=[
            pl.BlockSpec((1, Qn, Dn), lambda b, j: (b, 0, 0)),
            pl.BlockSpec((1, _TL, Dn), lambda b, j: (b, j, 0)),
            pl.BlockSpec((1, 1, Ln), lambda b, j: (b, 0, 0)),
            pl.BlockSpec((1, 1, _TL), lambda b, j: (b, 0, j)),
            pl.BlockSpec((Rn, Dn), lambda b, j: (0, 0)),
            pl.BlockSpec((1, Rn), lambda b, j: (0, 0)),
            pl.BlockSpec((Rn, Dn), lambda b, j: (0, 0)),
            pl.BlockSpec((1, Rn), lambda b, j: (0, 0)),
            pl.BlockSpec((Hn, Dn), lambda b, j: (0, 0)),
            pl.BlockSpec((1, Hn), lambda b, j: (0, 0)),
            pl.BlockSpec((1, Hn), lambda b, j: (0, 0)),
            pl.BlockSpec(memory_space=pltpu.SMEM),
        ],
        out_specs=(
            pl.BlockSpec((1, 1, _TL), lambda b, j: (b, 0, j)),
            pl.BlockSpec((1, 1, _TL), lambda b, j: (b, 0, j)),
            pl.BlockSpec((1, 1, 128), lambda b, j: (b, 0, 0)),
        ),
        out_shape=out_shape,
        compiler_params=pltpu.CompilerParams(
            dimension_semantics=("parallel", "arbitrary"),
        ),
    )(qe, ce, mask3, mask3, Wq, bq.reshape(1, Rn), Wc, bc.reshape(1, Rn),
      W1, b1.reshape(1, Hn), W2, b2.reshape(1, 1))
    return rel.reshape(Bn, Ln), keys.reshape(Bn, Ln), bud.reshape(Bn, 128)


def _select_body(keys_hbm, bud_hbm, out_hbm, keys_v, out_v, bud_v, L):
    nc = lax.axis_index("c")
    ns = lax.axis_index("s")
    wid = ns * 2 + nc
    pltpu.sync_copy(keys_hbm.at[wid], keys_v)
    pltpu.sync_copy(bud_hbm.at[wid], bud_v)
    k = lax.reduce_sum(bud_v[pl.ds(0, 16)], axes=(0,)) >> 4

    nchunk = L // 16
    group = 8

    def count_ge(thr):
        tv = jnp.full((16,), thr, jnp.uint32)

        def body(g, acc):
            for u in range(group):
                kv = keys_v[pl.ds((g * group + u) * 16, 16)]
                acc = acc + jnp.where(kv >= tv, 1, 0).astype(jnp.int32)
            return acc

        acc = lax.fori_loop(0, nchunk // group, body, jnp.zeros((16,), jnp.int32))
        return lax.reduce_sum(acc, axes=(0,))

    def bit_body(i, t):
        cand = t | (jnp.uint32(1) << (jnp.uint32(31) - i.astype(jnp.uint32)))
        c = count_ge(cand)
        return jnp.where(c >= k, cand, t)

    t = lax.fori_loop(0, 32, bit_body, jnp.uint32(0))
    tv = jnp.full((16,), t, jnp.uint32)

    def gt_body(g, acc):
        for u in range(group):
            kv = keys_v[pl.ds((g * group + u) * 16, 16)]
            acc = acc + jnp.where(kv > tv, 1, 0).astype(jnp.int32)
        return acc

    cnt_gt = lax.reduce_sum(
        lax.fori_loop(0, nchunk // group, gt_body, jnp.zeros((16,), jnp.int32)),
        axes=(0,))
    rem = k - cnt_gt

    def out_body(g, carry):
        for u in range(group):
            j = g * group + u
            kv = keys_v[pl.ds(j * 16, 16)]
            gt = kv > tv
            eq = kv == tv
            eqi = jnp.where(eq, 1, 0).astype(jnp.int32)
            pc = plsc.cumsum(eqi)
            sel = gt | (eq & ((carry + pc) <= rem))
            out_v[pl.ds(j * 16, 16)] = jnp.where(sel, 1.0, 0.0).astype(jnp.float32)
            carry = carry + lax.reduce_sum(eqi, axes=(0,))
        return carry

    lax.fori_loop(0, nchunk // group, out_body, jnp.int32(0))
    pltpu.sync_copy(out_v, out_hbm.at[wid])


def _select(keys, bud):
    Bn, Ln = keys.shape
    mesh = plsc.VectorSubcoreMesh(core_axis_name="c", subcore_axis_name="s")
    body = functools.partial(_select_body, L=Ln)
    return pl.kernel(
        body,
        mesh=mesh,
        out_type=jax.ShapeDtypeStruct((Bn, Ln), jnp.float32),
        scratch_types=[
            pltpu.VMEM((Ln,), jnp.uint32),
            pltpu.VMEM((Ln,), jnp.float32),
            pltpu.VMEM((128,), jnp.int32),
        ],
        compiler_params=pltpu.CompilerParams(needs_layout_passes=False),
    )(keys, bud)


def kernel(query_embeddings, context_embeddings, context_mask,
           Wq, bq, Wc, bc, W1, b1, W2, b2):
    rel, keys, bud = _dense(query_embeddings, context_embeddings, context_mask,
                            Wq, bq, Wc, bc, W1, b1, W2, b2)
    selection_mask = _select(keys, bud)
    return (selection_mask, rel)


# speedup vs baseline: 2.0447x; 1.1083x over previous
import functools

import jax
import jax.numpy as jnp
from jax import lax
from jax.experimental import pallas as pl
from jax.experimental.pallas import tpu as pltpu
from jax.experimental.pallas import tpu_sc as plsc

_BASE_BUDGET = 512.0
_ALPHA = 0.5
_TL = 512


def _dense_body(qe_ref, ce_ref, mask_ref, maskt_ref, wq_ref, bq_ref, wc_ref,
                bc_ref, w1_ref, b1_ref, w2_ref, b2_ref,
                rel_ref, keys_ref, bud_ref, qp_ref):
    j = pl.program_id(1)

    @pl.when(j == 0)
    def _query_path():
        qe = qe_ref[0]
        qr = lax.dot_general(qe, wq_ref[...], (((1,), (1,)), ((), ())),
                             preferred_element_type=jnp.float32) + bq_ref[...]
        qn = jnp.sqrt(jnp.sum(qr * qr, axis=1, keepdims=True))
        qr = qr / jnp.maximum(qn, 1e-12)
        qp = jnp.mean(qr, axis=0, keepdims=True)
        qpn = jnp.sqrt(jnp.sum(qp * qp, axis=1, keepdims=True))
        qp_ref[...] = qp / jnp.maximum(qpn, 1e-12)

        pooled = jnp.mean(qe, axis=0, keepdims=True)
        hidden = lax.dot_general(pooled, w1_ref[...], (((1,), (1,)), ((), ())),
                                 preferred_element_type=jnp.float32) + b1_ref[...]
        hidden = jnp.maximum(hidden, 0.0)
        logit = jnp.sum(hidden * w2_ref[...]) + b2_ref[0, 0]
        cx = 1.0 / (1.0 + jnp.exp(-logit))
        budf = jnp.round(_BASE_BUDGET * (1.0 + _ALPHA * cx))
        msum = jnp.sum(mask_ref[0])
        budm = jnp.minimum(budf, msum)
        bud_ref[0] = jnp.full((1, 128), budm, jnp.float32).astype(jnp.int32)

    ce = ce_ref[0]
    raw = lax.dot_general(ce, wc_ref[...], (((1,), (1,)), ((), ())),
                          preferred_element_type=jnp.float32)
    raw = raw + bc_ref[...]

    n = jnp.sqrt(jnp.sum(raw * raw, axis=1, keepdims=True))
    chat = raw / jnp.maximum(n, 1e-12)
    rel = lax.dot_general(qp_ref[...], chat, (((1,), (1,)), ((), ())),
                          preferred_element_type=jnp.float32)

    mt = maskt_ref[0]
    rel = jnp.where(mt > 0.0, rel, -jnp.inf)
    rel_ref[0] = rel

    u = lax.bitcast_convert_type(rel, jnp.uint32)
    key = u ^ jnp.where(u >> 31 == 1, jnp.uint32(0xFFFFFFFF), jnp.uint32(0x80000000))
    keys_ref[0] = key


def _dense(qe, ce, mask, Wq, bq, Wc, bc, W1, b1, W2, b2):
    Bn, Qn, Dn = qe.shape
    Ln = ce.shape[1]
    Rn = Wq.shape[0]
    Hn = W1.shape[0]
    grid = (Bn, Ln // _TL)
    mask3 = mask.reshape(Bn, 1, Ln)
    out_shape = (
        jax.ShapeDtypeStruct((Bn, 1, Ln), jnp.float32),
        jax.ShapeDtypeStruct((Bn, 1, Ln), jnp.uint32),
        jax.ShapeDtypeStruct((Bn, 1, 128), jnp.int32),
    )
    rel, keys, bud = pl.pallas_call(
        _dense_body,
        grid=grid,
        in_specs=[
            pl.BlockSpec((1, Qn, Dn), lambda b, j: (b, 0, 0)),
            pl.BlockSpec((1, _TL, Dn), lambda b, j: (b, j, 0)),
            pl.BlockSpec((1, 1, Ln), lambda b, j: (b, 0, 0)),
            pl.BlockSpec((1, 1, _TL), lambda b, j: (b, 0, j)),
            pl.BlockSpec((Rn, Dn), lambda b, j: (0, 0)),
            pl.BlockSpec((1, Rn), lambda b, j: (0, 0)),
            pl.BlockSpec((Rn, Dn), lambda b, j: (0, 0)),
            pl.BlockSpec((1, Rn), lambda b, j: (0, 0)),
            pl.BlockSpec((Hn, Dn), lambda b, j: (0, 0)),
            pl.BlockSpec((1, Hn), lambda b, j: (0, 0)),
            pl.BlockSpec((1, Hn), lambda b, j: (0, 0)),
            pl.BlockSpec(memory_space=pltpu.SMEM),
        ],
        out_specs=(
            pl.BlockSpec((1, 1, _TL), lambda b, j: (b, 0, j)),
            pl.BlockSpec((1, 1, _TL), lambda b, j: (b, 0, j)),
            pl.BlockSpec((1, 1, 128), lambda b, j: (b, 0, 0)),
        ),
        out_shape=out_shape,
        scratch_shapes=[pltpu.VMEM((1, Rn), jnp.float32)],
        compiler_params=pltpu.CompilerParams(
            dimension_semantics=("parallel", "arbitrary"),
        ),
    )(qe, ce, mask3, mask3, Wq, bq.reshape(1, Rn), Wc, bc.reshape(1, Rn),
      W1, b1.reshape(1, Hn), W2, b2.reshape(1, 1))
    return rel.reshape(Bn, Ln), keys.reshape(Bn, Ln), bud.reshape(Bn, 128)


def _select_body(keys_hbm, bud_hbm, out_hbm, keys_v, out_v, bud_v, L):
    nc = lax.axis_index("c")
    ns = lax.axis_index("s")
    wid = ns * 2 + nc
    pltpu.sync_copy(keys_hbm.at[wid], keys_v)
    pltpu.sync_copy(bud_hbm.at[wid], bud_v)
    k = lax.reduce_sum(bud_v[pl.ds(0, 16)], axes=(0,)) >> 4

    nchunk = L // 16
    group = 8

    def count_ge(thr):
        tv = jnp.full((16,), thr, jnp.uint32)

        def body(g, acc):
            for u in range(group):
                kv = keys_v[pl.ds((g * group + u) * 16, 16)]
                acc = acc + jnp.where(kv >= tv, 1, 0).astype(jnp.int32)
            return acc

        acc = lax.fori_loop(0, nchunk // group, body, jnp.zeros((16,), jnp.int32))
        return lax.reduce_sum(acc, axes=(0,))

    def bit_body(i, t):
        cand = t | (jnp.uint32(1) << (jnp.uint32(31) - i.astype(jnp.uint32)))
        c = count_ge(cand)
        return jnp.where(c >= k, cand, t)

    t = lax.fori_loop(0, 32, bit_body, jnp.uint32(0))
    tv = jnp.full((16,), t, jnp.uint32)

    def gt_body(g, acc):
        for u in range(group):
            kv = keys_v[pl.ds((g * group + u) * 16, 16)]
            acc = acc + jnp.where(kv > tv, 1, 0).astype(jnp.int32)
        return acc

    cnt_gt = lax.reduce_sum(
        lax.fori_loop(0, nchunk // group, gt_body, jnp.zeros((16,), jnp.int32)),
        axes=(0,))
    rem = k - cnt_gt

    def out_body(g, carry):
        for u in range(group):
            j = g * group + u
            kv = keys_v[pl.ds(j * 16, 16)]
            gt = kv > tv
            eq = kv == tv
            eqi = jnp.where(eq, 1, 0).astype(jnp.int32)
            pc = plsc.cumsum(eqi)
            sel = gt | (eq & ((carry + pc) <= rem))
            out_v[pl.ds(j * 16, 16)] = jnp.where(sel, 1.0, 0.0).astype(jnp.float32)
            carry = carry + lax.reduce_sum(eqi, axes=(0,))
        return carry

    lax.fori_loop(0, nchunk // group, out_body, jnp.int32(0))
    pltpu.sync_copy(out_v, out_hbm.at[wid])


def _select(keys, bud):
    Bn, Ln = keys.shape
    mesh = plsc.VectorSubcoreMesh(core_axis_name="c", subcore_axis_name="s")
    body = functools.partial(_select_body, L=Ln)
    return pl.kernel(
        body,
        mesh=mesh,
        out_type=jax.ShapeDtypeStruct((Bn, Ln), jnp.float32),
        scratch_types=[
            pltpu.VMEM((Ln,), jnp.uint32),
            pltpu.VMEM((Ln,), jnp.float32),
            pltpu.VMEM((128,), jnp.int32),
        ],
        compiler_params=pltpu.CompilerParams(needs_layout_passes=False),
    )(keys, bud)


def kernel(query_embeddings, context_embeddings, context_mask,
           Wq, bq, Wc, bc, W1, b1, W2, b2):
    rel, keys, bud = _dense(query_embeddings, context_embeddings, context_mask,
                            Wq, bq, Wc, bc, W1, b1, W2, b2)
    selection_mask = _select(keys, bud)
    return (selection_mask, rel)


# speedup vs baseline: 2.7884x; 1.3638x over previous
import functools

import jax
import jax.numpy as jnp
from jax import lax
from jax.experimental import pallas as pl
from jax.experimental.pallas import tpu as pltpu
from jax.experimental.pallas import tpu_sc as plsc

_BASE_BUDGET = 512.0
_ALPHA = 0.5
_TL = 1024


def _dense_body(qe_ref, ce_ref, mask_ref, maskt_ref, wq_ref, bq_ref, wc_ref,
                bc_ref, w1_ref, b1_ref, w2_ref, b2_ref,
                rel_ref, keys_ref, bud_ref, qp_ref):
    j = pl.program_id(1)

    @pl.when(j == 0)
    def _query_path():
        qe = qe_ref[0]
        qr = lax.dot_general(qe, wq_ref[...], (((1,), (1,)), ((), ())),
                             preferred_element_type=jnp.float32) + bq_ref[...]
        qn = jnp.sqrt(jnp.sum(qr * qr, axis=1, keepdims=True))
        qr = qr / jnp.maximum(qn, 1e-12)
        qp = jnp.mean(qr, axis=0, keepdims=True)
        qpn = jnp.sqrt(jnp.sum(qp * qp, axis=1, keepdims=True))
        qp_ref[...] = qp / jnp.maximum(qpn, 1e-12)

        pooled = jnp.mean(qe, axis=0, keepdims=True)
        hidden = lax.dot_general(pooled, w1_ref[...], (((1,), (1,)), ((), ())),
                                 preferred_element_type=jnp.float32) + b1_ref[...]
        hidden = jnp.maximum(hidden, 0.0)
        logit = jnp.sum(hidden * w2_ref[...]) + b2_ref[0, 0]
        cx = 1.0 / (1.0 + jnp.exp(-logit))
        budf = jnp.round(_BASE_BUDGET * (1.0 + _ALPHA * cx))
        msum = jnp.sum(mask_ref[0])
        budm = jnp.minimum(budf, msum)
        bud_ref[0] = jnp.full((1, 128), budm, jnp.float32).astype(jnp.int32)

    ce = ce_ref[0]
    raw = lax.dot_general(ce, wc_ref[...], (((1,), (1,)), ((), ())),
                          preferred_element_type=jnp.float32)
    raw = raw + bc_ref[...]

    n = jnp.sqrt(jnp.sum(raw * raw, axis=1, keepdims=True))
    chat = raw / jnp.maximum(n, 1e-12)
    rel = lax.dot_general(qp_ref[...], chat, (((1,), (1,)), ((), ())),
                          preferred_element_type=jnp.float32)

    mt = maskt_ref[0]
    rel = jnp.where(mt > 0.0, rel, -jnp.inf)
    rel_ref[0] = rel

    u = lax.bitcast_convert_type(rel, jnp.uint32)
    key = u ^ jnp.where(u >> 31 == 1, jnp.uint32(0xFFFFFFFF), jnp.uint32(0x80000000))
    keys_ref[0] = key


def _dense(qe, ce, mask, Wq, bq, Wc, bc, W1, b1, W2, b2):
    Bn, Qn, Dn = qe.shape
    Ln = ce.shape[1]
    Rn = Wq.shape[0]
    Hn = W1.shape[0]
    grid = (Bn, Ln // _TL)
    mask3 = mask.reshape(Bn, 1, Ln)
    out_shape = (
        jax.ShapeDtypeStruct((Bn, 1, Ln), jnp.float32),
        jax.ShapeDtypeStruct((Bn, 1, Ln), jnp.uint32),
        jax.ShapeDtypeStruct((Bn, 1, 128), jnp.int32),
    )
    rel, keys, bud = pl.pallas_call(
        _dense_body,
        grid=grid,
        in_specs=[
            pl.BlockSpec((1, Qn, Dn), lambda b, j: (b, 0, 0)),
            pl.BlockSpec((1, _TL, Dn), lambda b, j: (b, j, 0)),
            pl.BlockSpec((1, 1, Ln), lambda b, j: (b, 0, 0)),
            pl.BlockSpec((1, 1, _TL), lambda b, j: (b, 0, j)),
            pl.BlockSpec((Rn, Dn), lambda b, j: (0, 0)),
            pl.BlockSpec((1, Rn), lambda b, j: (0, 0)),
            pl.BlockSpec((Rn, Dn), lambda b, j: (0, 0)),
            pl.BlockSpec((1, Rn), lambda b, j: (0, 0)),
            pl.BlockSpec((Hn, Dn), lambda b, j: (0, 0)),
            pl.BlockSpec((1, Hn), lambda b, j: (0, 0)),
            pl.BlockSpec((1, Hn), lambda b, j: (0, 0)),
            pl.BlockSpec(memory_space=pltpu.SMEM),
        ],
        out_specs=(
            pl.BlockSpec((1, 1, _TL), lambda b, j: (b, 0, j)),
            pl.BlockSpec((1, 1, _TL), lambda b, j: (b, 0, j)),
            pl.BlockSpec((1, 1, 128), lambda b, j: (b, 0, 0)),
        ),
        out_shape=out_shape,
        scratch_shapes=[pltpu.VMEM((1, Rn), jnp.float32)],
        compiler_params=pltpu.CompilerParams(
            dimension_semantics=("parallel", "arbitrary"),
        ),
    )(qe, ce, mask3, mask3, Wq, bq.reshape(1, Rn), Wc, bc.reshape(1, Rn),
      W1, b1.reshape(1, Hn), W2, b2.reshape(1, 1))
    return rel.reshape(Bn, Ln), keys.reshape(Bn, Ln), bud.reshape(Bn, 128)


def _select_body(keys_hbm, bud_hbm, out_hbm, keys_v, out_v, bud_v, L):
    nc = lax.axis_index("c")
    ns = lax.axis_index("s")
    wid = ns * 2 + nc
    pltpu.sync_copy(keys_hbm.at[wid], keys_v)
    pltpu.sync_copy(bud_hbm.at[wid], bud_v)
    k = lax.reduce_sum(bud_v[pl.ds(0, 16)], axes=(0,)) >> 4

    nchunk = L // 16
    group = 8

    def count_ge(thr):
        tv = jnp.full((16,), thr, jnp.uint32)

        def body(g, acc):
            for u in range(group):
                kv = keys_v[pl.ds((g * group + u) * 16, 16)]
                acc = acc + jnp.where(kv >= tv, 1, 0).astype(jnp.int32)
            return acc

        acc = lax.fori_loop(0, nchunk // group, body, jnp.zeros((16,), jnp.int32))
        return lax.reduce_sum(acc, axes=(0,))

    def bit_body(i, t):
        cand = t | (jnp.uint32(1) << (jnp.uint32(31) - i.astype(jnp.uint32)))
        c = count_ge(cand)
        return jnp.where(c >= k, cand, t)

    t = lax.fori_loop(0, 32, bit_body, jnp.uint32(0))
    tv = jnp.full((16,), t, jnp.uint32)

    def gt_body(g, acc):
        for u in range(group):
            kv = keys_v[pl.ds((g * group + u) * 16, 16)]
            acc = acc + jnp.where(kv > tv, 1, 0).astype(jnp.int32)
        return acc

    cnt_gt = lax.reduce_sum(
        lax.fori_loop(0, nchunk // group, gt_body, jnp.zeros((16,), jnp.int32)),
        axes=(0,))
    rem = k - cnt_gt

    def out_body(g, carry):
        for u in range(group):
            j = g * group + u
            kv = keys_v[pl.ds(j * 16, 16)]
            gt = kv > tv
            eq = kv == tv
            eqi = jnp.where(eq, 1, 0).astype(jnp.int32)
            pc = plsc.cumsum(eqi)
            sel = gt | (eq & ((carry + pc) <= rem))
            out_v[pl.ds(j * 16, 16)] = jnp.where(sel, 1.0, 0.0).astype(jnp.float32)
            carry = carry + lax.reduce_sum(eqi, axes=(0,))
        return carry

    lax.fori_loop(0, nchunk // group, out_body, jnp.int32(0))
    pltpu.sync_copy(out_v, out_hbm.at[wid])


def _select(keys, bud):
    Bn, Ln = keys.shape
    mesh = plsc.VectorSubcoreMesh(core_axis_name="c", subcore_axis_name="s")
    body = functools.partial(_select_body, L=Ln)
    return pl.kernel(
        body,
        mesh=mesh,
        out_type=jax.ShapeDtypeStruct((Bn, Ln), jnp.float32),
        scratch_types=[
            pltpu.VMEM((Ln,), jnp.uint32),
            pltpu.VMEM((Ln,), jnp.float32),
            pltpu.VMEM((128,), jnp.int32),
        ],
        compiler_params=pltpu.CompilerParams(needs_layout_passes=False),
    )(keys, bud)


def kernel(query_embeddings, context_embeddings, context_mask,
           Wq, bq, Wc, bc, W1, b1, W2, b2):
    rel, keys, bud = _dense(query_embeddings, context_embeddings, context_mask,
                            Wq, bq, Wc, bc, W1, b1, W2, b2)
    selection_mask = _select(keys, bud)
    return (selection_mask, rel)


# speedup vs baseline: 3.4009x; 1.2196x over previous
import functools

import jax
import jax.numpy as jnp
from jax import lax
from jax.experimental import pallas as pl
from jax.experimental.pallas import tpu as pltpu
from jax.experimental.pallas import tpu_sc as plsc

_BASE_BUDGET = 512.0
_ALPHA = 0.5
_TL = 2048


def _dense_body(qe_ref, ce_ref, mask_ref, maskt_ref, wq_ref, bq_ref, wc_ref,
                bc_ref, w1_ref, b1_ref, w2_ref, b2_ref,
                rel_ref, keys_ref, bud_ref, qp_ref):
    j = pl.program_id(1)

    @pl.when(j == 0)
    def _query_path():
        qe = qe_ref[0]
        qr = lax.dot_general(qe, wq_ref[...], (((1,), (1,)), ((), ())),
                             preferred_element_type=jnp.float32) + bq_ref[...]
        qn = jnp.sqrt(jnp.sum(qr * qr, axis=1, keepdims=True))
        qr = qr / jnp.maximum(qn, 1e-12)
        qp = jnp.mean(qr, axis=0, keepdims=True)
        qpn = jnp.sqrt(jnp.sum(qp * qp, axis=1, keepdims=True))
        qp_ref[...] = qp / jnp.maximum(qpn, 1e-12)

        pooled = jnp.mean(qe, axis=0, keepdims=True)
        hidden = lax.dot_general(pooled, w1_ref[...], (((1,), (1,)), ((), ())),
                                 preferred_element_type=jnp.float32) + b1_ref[...]
        hidden = jnp.maximum(hidden, 0.0)
        logit = jnp.sum(hidden * w2_ref[...]) + b2_ref[0, 0]
        cx = 1.0 / (1.0 + jnp.exp(-logit))
        budf = jnp.round(_BASE_BUDGET * (1.0 + _ALPHA * cx))
        msum = jnp.sum(mask_ref[0])
        budm = jnp.minimum(budf, msum)
        bud_ref[0] = jnp.full((1, 128), budm, jnp.float32).astype(jnp.int32)

    ce = ce_ref[0]
    raw = lax.dot_general(ce, wc_ref[...], (((1,), (1,)), ((), ())),
                          preferred_element_type=jnp.float32)
    raw = raw + bc_ref[...]

    n = jnp.sqrt(jnp.sum(raw * raw, axis=1, keepdims=True))
    chat = raw / jnp.maximum(n, 1e-12)
    rel = lax.dot_general(qp_ref[...], chat, (((1,), (1,)), ((), ())),
                          preferred_element_type=jnp.float32)

    mt = maskt_ref[0]
    rel = jnp.where(mt > 0.0, rel, -jnp.inf)
    rel_ref[0] = rel

    u = lax.bitcast_convert_type(rel, jnp.uint32)
    key = u ^ jnp.where(u >> 31 == 1, jnp.uint32(0xFFFFFFFF), jnp.uint32(0x80000000))
    keys_ref[0] = key


def _dense(qe, ce, mask, Wq, bq, Wc, bc, W1, b1, W2, b2):
    Bn, Qn, Dn = qe.shape
    Ln = ce.shape[1]
    Rn = Wq.shape[0]
    Hn = W1.shape[0]
    grid = (Bn, Ln // _TL)
    mask3 = mask.reshape(Bn, 1, Ln)
    out_shape = (
        jax.ShapeDtypeStruct((Bn, 1, Ln), jnp.float32),
        jax.ShapeDtypeStruct((Bn, 1, Ln), jnp.uint32),
        jax.ShapeDtypeStruct((Bn, 1, 128), jnp.int32),
    )
    rel, keys, bud = pl.pallas_call(
        _dense_body,
        grid=grid,
        in_specs=[
            pl.BlockSpec((1, Qn, Dn), lambda b, j: (b, 0, 0)),
            pl.BlockSpec((1, _TL, Dn), lambda b, j: (b, j, 0)),
            pl.BlockSpec((1, 1, Ln), lambda b, j: (b, 0, 0)),
            pl.BlockSpec((1, 1, _TL), lambda b, j: (b, 0, j)),
            pl.BlockSpec((Rn, Dn), lambda b, j: (0, 0)),
            pl.BlockSpec((1, Rn), lambda b, j: (0, 0)),
            pl.BlockSpec((Rn, Dn), lambda b, j: (0, 0)),
            pl.BlockSpec((1, Rn), lambda b, j: (0, 0)),
            pl.BlockSpec((Hn, Dn), lambda b, j: (0, 0)),
            pl.BlockSpec((1, Hn), lambda b, j: (0, 0)),
            pl.BlockSpec((1, Hn), lambda b, j: (0, 0)),
            pl.BlockSpec(memory_space=pltpu.SMEM),
        ],
        out_specs=(
            pl.BlockSpec((1, 1, _TL), lambda b, j: (b, 0, j)),
            pl.BlockSpec((1, 1, _TL), lambda b, j: (b, 0, j)),
            pl.BlockSpec((1, 1, 128), lambda b, j: (b, 0, 0)),
        ),
        out_shape=out_shape,
        scratch_shapes=[pltpu.VMEM((1, Rn), jnp.float32)],
        compiler_params=pltpu.CompilerParams(
            dimension_semantics=("parallel", "arbitrary"),
        ),
    )(qe, ce, mask3, mask3, Wq, bq.reshape(1, Rn), Wc, bc.reshape(1, Rn),
      W1, b1.reshape(1, Hn), W2, b2.reshape(1, 1))
    return rel.reshape(Bn, Ln), keys.reshape(Bn, Ln), bud.reshape(Bn, 128)


def _select_body(keys_hbm, bud_hbm, out_hbm, keys_v, out_v, bud_v, L):
    nc = lax.axis_index("c")
    ns = lax.axis_index("s")
    wid = ns * 2 + nc
    pltpu.sync_copy(keys_hbm.at[wid], keys_v)
    pltpu.sync_copy(bud_hbm.at[wid], bud_v)
    k = lax.reduce_sum(bud_v[pl.ds(0, 16)], axes=(0,)) >> 4

    nchunk = L // 16
    group = 8

    def count_ge(thr):
        tv = jnp.full((16,), thr, jnp.uint32)

        def body(g, acc):
            for u in range(group):
                kv = keys_v[pl.ds((g * group + u) * 16, 16)]
                acc = acc + jnp.where(kv >= tv, 1, 0).astype(jnp.int32)
            return acc

        acc = lax.fori_loop(0, nchunk // group, body, jnp.zeros((16,), jnp.int32))
        return lax.reduce_sum(acc, axes=(0,))

    def bit_body(i, t):
        cand = t | (jnp.uint32(1) << (jnp.uint32(31) - i.astype(jnp.uint32)))
        c = count_ge(cand)
        return jnp.where(c >= k, cand, t)

    t = lax.fori_loop(0, 32, bit_body, jnp.uint32(0))
    tv = jnp.full((16,), t, jnp.uint32)

    def gt_body(g, acc):
        for u in range(group):
            kv = keys_v[pl.ds((g * group + u) * 16, 16)]
            acc = acc + jnp.where(kv > tv, 1, 0).astype(jnp.int32)
        return acc

    cnt_gt = lax.reduce_sum(
        lax.fori_loop(0, nchunk // group, gt_body, jnp.zeros((16,), jnp.int32)),
        axes=(0,))
    rem = k - cnt_gt

    def out_body(g, carry):
        for u in range(group):
            j = g * group + u
            kv = keys_v[pl.ds(j * 16, 16)]
            gt = kv > tv
            eq = kv == tv
            eqi = jnp.where(eq, 1, 0).astype(jnp.int32)
            pc = plsc.cumsum(eqi)
            sel = gt | (eq & ((carry + pc) <= rem))
            out_v[pl.ds(j * 16, 16)] = jnp.where(sel, 1.0, 0.0).astype(jnp.float32)
            carry = carry + lax.reduce_sum(eqi, axes=(0,))
        return carry

    lax.fori_loop(0, nchunk // group, out_body, jnp.int32(0))
    pltpu.sync_copy(out_v, out_hbm.at[wid])


def _select(keys, bud):
    Bn, Ln = keys.shape
    mesh = plsc.VectorSubcoreMesh(core_axis_name="c", subcore_axis_name="s")
    body = functools.partial(_select_body, L=Ln)
    return pl.kernel(
        body,
        mesh=mesh,
        out_type=jax.ShapeDtypeStruct((Bn, Ln), jnp.float32),
        scratch_types=[
            pltpu.VMEM((Ln,), jnp.uint32),
            pltpu.VMEM((Ln,), jnp.float32),
            pltpu.VMEM((128,), jnp.int32),
        ],
        compiler_params=pltpu.CompilerParams(needs_layout_passes=False),
    )(keys, bud)


def kernel(query_embeddings, context_embeddings, context_mask,
           Wq, bq, Wc, bc, W1, b1, W2, b2):
    rel, keys, bud = _dense(query_embeddings, context_embeddings, context_mask,
                            Wq, bq, Wc, bc, W1, b1, W2, b2)
    selection_mask = _select(keys, bud)
    return (selection_mask, rel)


# speedup vs baseline: 3.7446x; 1.1011x over previous
import functools

import jax
import jax.numpy as jnp
from jax import lax
from jax.experimental import pallas as pl
from jax.experimental.pallas import tpu as pltpu
from jax.experimental.pallas import tpu_sc as plsc

_BASE_BUDGET = 512.0
_ALPHA = 0.5
_TL = 4096


def _dense_body(qe_ref, ce_ref, mask_ref, maskt_ref, wq_ref, bq_ref, wc_ref,
                bc_ref, w1_ref, b1_ref, w2_ref, b2_ref,
                rel_ref, keys_ref, bud_ref, qp_ref):
    j = pl.program_id(1)

    @pl.when(j == 0)
    def _query_path():
        qe = qe_ref[0]
        qr = lax.dot_general(qe, wq_ref[...], (((1,), (1,)), ((), ())),
                             preferred_element_type=jnp.float32) + bq_ref[...]
        qn = jnp.sqrt(jnp.sum(qr * qr, axis=1, keepdims=True))
        qr = qr / jnp.maximum(qn, 1e-12)
        qp = jnp.mean(qr, axis=0, keepdims=True)
        qpn = jnp.sqrt(jnp.sum(qp * qp, axis=1, keepdims=True))
        qp_ref[...] = qp / jnp.maximum(qpn, 1e-12)

        pooled = jnp.mean(qe, axis=0, keepdims=True)
        hidden = lax.dot_general(pooled, w1_ref[...], (((1,), (1,)), ((), ())),
                                 preferred_element_type=jnp.float32) + b1_ref[...]
        hidden = jnp.maximum(hidden, 0.0)
        logit = jnp.sum(hidden * w2_ref[...]) + b2_ref[0, 0]
        cx = 1.0 / (1.0 + jnp.exp(-logit))
        budf = jnp.round(_BASE_BUDGET * (1.0 + _ALPHA * cx))
        msum = jnp.sum(mask_ref[0])
        budm = jnp.minimum(budf, msum)
        bud_ref[0] = jnp.full((1, 128), budm, jnp.float32).astype(jnp.int32)

    ce = ce_ref[0]
    raw = lax.dot_general(ce, wc_ref[...], (((1,), (1,)), ((), ())),
                          preferred_element_type=jnp.float32)
    raw = raw + bc_ref[...]

    n = jnp.sqrt(jnp.sum(raw * raw, axis=1, keepdims=True))
    chat = raw / jnp.maximum(n, 1e-12)
    rel = lax.dot_general(qp_ref[...], chat, (((1,), (1,)), ((), ())),
                          preferred_element_type=jnp.float32)

    mt = maskt_ref[0]
    rel = jnp.where(mt > 0.0, rel, -jnp.inf)
    rel_ref[0] = rel

    u = lax.bitcast_convert_type(rel, jnp.uint32)
    key = u ^ jnp.where(u >> 31 == 1, jnp.uint32(0xFFFFFFFF), jnp.uint32(0x80000000))
    keys_ref[0] = key


def _dense(qe, ce, mask, Wq, bq, Wc, bc, W1, b1, W2, b2):
    Bn, Qn, Dn = qe.shape
    Ln = ce.shape[1]
    Rn = Wq.shape[0]
    Hn = W1.shape[0]
    grid = (Bn, Ln // _TL)
    mask3 = mask.reshape(Bn, 1, Ln)
    out_shape = (
        jax.ShapeDtypeStruct((Bn, 1, Ln), jnp.float32),
        jax.ShapeDtypeStruct((Bn, 1, Ln), jnp.uint32),
        jax.ShapeDtypeStruct((Bn, 1, 128), jnp.int32),
    )
    rel, keys, bud = pl.pallas_call(
        _dense_body,
        grid=grid,
        in_specs=[
            pl.BlockSpec((1, Qn, Dn), lambda b, j: (b, 0, 0)),
            pl.BlockSpec((1, _TL, Dn), lambda b, j: (b, j, 0)),
            pl.BlockSpec((1, 1, Ln), lambda b, j: (b, 0, 0)),
            pl.BlockSpec((1, 1, _TL), lambda b, j: (b, 0, j)),
            pl.BlockSpec((Rn, Dn), lambda b, j: (0, 0)),
            pl.BlockSpec((1, Rn), lambda b, j: (0, 0)),
            pl.BlockSpec((Rn, Dn), lambda b, j: (0, 0)),
            pl.BlockSpec((1, Rn), lambda b, j: (0, 0)),
            pl.BlockSpec((Hn, Dn), lambda b, j: (0, 0)),
            pl.BlockSpec((1, Hn), lambda b, j: (0, 0)),
            pl.BlockSpec((1, Hn), lambda b, j: (0, 0)),
            pl.BlockSpec(memory_space=pltpu.SMEM),
        ],
        out_specs=(
            pl.BlockSpec((1, 1, _TL), lambda b, j: (b, 0, j)),
            pl.BlockSpec((1, 1, _TL), lambda b, j: (b, 0, j)),
            pl.BlockSpec((1, 1, 128), lambda b, j: (b, 0, 0)),
        ),
        out_shape=out_shape,
        scratch_shapes=[pltpu.VMEM((1, Rn), jnp.float32)],
        compiler_params=pltpu.CompilerParams(
            dimension_semantics=("parallel", "arbitrary"),
        ),
    )(qe, ce, mask3, mask3, Wq, bq.reshape(1, Rn), Wc, bc.reshape(1, Rn),
      W1, b1.reshape(1, Hn), W2, b2.reshape(1, 1))
    return rel.reshape(Bn, Ln), keys.reshape(Bn, Ln), bud.reshape(Bn, 128)


def _select_body(keys_hbm, bud_hbm, out_hbm, keys_v, out_v, bud_v, L):
    nc = lax.axis_index("c")
    ns = lax.axis_index("s")
    wid = ns * 2 + nc
    pltpu.sync_copy(keys_hbm.at[wid], keys_v)
    pltpu.sync_copy(bud_hbm.at[wid], bud_v)
    k = lax.reduce_sum(bud_v[pl.ds(0, 16)], axes=(0,)) >> 4

    nchunk = L // 16
    group = 8

    def count_ge(thr):
        tv = jnp.full((16,), thr, jnp.uint32)

        def body(g, acc):
            for u in range(group):
                kv = keys_v[pl.ds((g * group + u) * 16, 16)]
                acc = acc + jnp.where(kv >= tv, 1, 0).astype(jnp.int32)
            return acc

        acc = lax.fori_loop(0, nchunk // group, body, jnp.zeros((16,), jnp.int32))
        return lax.reduce_sum(acc, axes=(0,))

    def bit_body(i, t):
        cand = t | (jnp.uint32(1) << (jnp.uint32(31) - i.astype(jnp.uint32)))
        c = count_ge(cand)
        return jnp.where(c >= k, cand, t)

    t = lax.fori_loop(0, 32, bit_body, jnp.uint32(0))
    tv = jnp.full((16,), t, jnp.uint32)

    def gt_body(g, acc):
        for u in range(group):
            kv = keys_v[pl.ds((g * group + u) * 16, 16)]
            acc = acc + jnp.where(kv > tv, 1, 0).astype(jnp.int32)
        return acc

    cnt_gt = lax.reduce_sum(
        lax.fori_loop(0, nchunk // group, gt_body, jnp.zeros((16,), jnp.int32)),
        axes=(0,))
    rem = k - cnt_gt

    def out_body(g, carry):
        for u in range(group):
            j = g * group + u
            kv = keys_v[pl.ds(j * 16, 16)]
            gt = kv > tv
            eq = kv == tv
            eqi = jnp.where(eq, 1, 0).astype(jnp.int32)
            pc = plsc.cumsum(eqi)
            sel = gt | (eq & ((carry + pc) <= rem))
            out_v[pl.ds(j * 16, 16)] = jnp.where(sel, 1.0, 0.0).astype(jnp.float32)
            carry = carry + lax.reduce_sum(eqi, axes=(0,))
        return carry

    lax.fori_loop(0, nchunk // group, out_body, jnp.int32(0))
    pltpu.sync_copy(out_v, out_hbm.at[wid])


def _select(keys, bud):
    Bn, Ln = keys.shape
    mesh = plsc.VectorSubcoreMesh(core_axis_name="c", subcore_axis_name="s")
    body = functools.partial(_select_body, L=Ln)
    return pl.kernel(
        body,
        mesh=mesh,
        out_type=jax.ShapeDtypeStruct((Bn, Ln), jnp.float32),
        scratch_types=[
            pltpu.VMEM((Ln,), jnp.uint32),
            pltpu.VMEM((Ln,), jnp.float32),
            pltpu.VMEM((128,), jnp.int32),
        ],
        compiler_params=pltpu.CompilerParams(needs_layout_passes=False),
    )(keys, bud)


def kernel(query_embeddings, context_embeddings, context_mask,
           Wq, bq, Wc, bc, W1, b1, W2, b2):
    rel, keys, bud = _dense(query_embeddings, context_embeddings, context_mask,
                            Wq, bq, Wc, bc, W1, b1, W2, b2)
    selection_mask = _select(keys, bud)
    return (selection_mask, rel)


# speedup vs baseline: 3.7642x; 1.0052x over previous
import functools

import jax
import jax.numpy as jnp
from jax import lax
from jax.experimental import pallas as pl
from jax.experimental.pallas import tpu as pltpu
from jax.experimental.pallas import tpu_sc as plsc

_BASE_BUDGET = 512.0
_ALPHA = 0.5
_TL = 4096


def _dense_body(qe_ref, ce_ref, mask_ref, maskt_ref, wq_ref, bq_ref, wc_ref,
                bc_ref, w1_ref, b1_ref, w2_ref, b2_ref,
                rel_ref, keys_ref, bud_ref, qp_ref):
    j = pl.program_id(1)

    @pl.when(j == 0)
    def _query_path():
        qe = qe_ref[0]
        qr = lax.dot_general(qe, wq_ref[...], (((1,), (1,)), ((), ())),
                             preferred_element_type=jnp.float32) + bq_ref[...]
        qn = jnp.sqrt(jnp.sum(qr * qr, axis=1, keepdims=True))
        qr = qr / jnp.maximum(qn, 1e-12)
        qp = jnp.mean(qr, axis=0, keepdims=True)
        qpn = jnp.sqrt(jnp.sum(qp * qp, axis=1, keepdims=True))
        qp_ref[...] = qp / jnp.maximum(qpn, 1e-12)

        pooled = jnp.mean(qe, axis=0, keepdims=True)
        hidden = lax.dot_general(pooled, w1_ref[...], (((1,), (1,)), ((), ())),
                                 preferred_element_type=jnp.float32) + b1_ref[...]
        hidden = jnp.maximum(hidden, 0.0)
        logit = jnp.sum(hidden * w2_ref[...]) + b2_ref[0, 0]
        cx = 1.0 / (1.0 + jnp.exp(-logit))
        budf = jnp.round(_BASE_BUDGET * (1.0 + _ALPHA * cx))
        msum = jnp.sum(mask_ref[0])
        budm = jnp.minimum(budf, msum)
        bud_ref[0] = jnp.full((1, 128), budm, jnp.float32).astype(jnp.int32)

    ce = ce_ref[0]
    wc = wc_ref[...]
    raw = lax.dot_general(ce[:, :256], wc[:, :256], (((1,), (1,)), ((), ())),
                          preferred_element_type=jnp.float32)
    for i in range(1, 4):
        raw = raw + lax.dot_general(ce[:, i * 256:(i + 1) * 256],
                                    wc[:, i * 256:(i + 1) * 256],
                                    (((1,), (1,)), ((), ())),
                                    preferred_element_type=jnp.float32)
    raw = raw + bc_ref[...]

    n = jnp.sqrt(jnp.sum(raw * raw, axis=1, keepdims=True))
    chat = raw / jnp.maximum(n, 1e-12)
    rel = lax.dot_general(qp_ref[...], chat, (((1,), (1,)), ((), ())),
                          preferred_element_type=jnp.float32)

    mt = maskt_ref[0]
    rel = jnp.where(mt > 0.0, rel, -jnp.inf)
    rel_ref[0] = rel

    u = lax.bitcast_convert_type(rel, jnp.uint32)
    key = u ^ jnp.where(u >> 31 == 1, jnp.uint32(0xFFFFFFFF), jnp.uint32(0x80000000))
    keys_ref[0] = key


def _dense(qe, ce, mask, Wq, bq, Wc, bc, W1, b1, W2, b2):
    Bn, Qn, Dn = qe.shape
    Ln = ce.shape[1]
    Rn = Wq.shape[0]
    Hn = W1.shape[0]
    grid = (Bn, Ln // _TL)
    mask3 = mask.reshape(Bn, 1, Ln)
    out_shape = (
        jax.ShapeDtypeStruct((Bn, 1, Ln), jnp.float32),
        jax.ShapeDtypeStruct((Bn, 1, Ln), jnp.uint32),
        jax.ShapeDtypeStruct((Bn, 1, 128), jnp.int32),
    )
    rel, keys, bud = pl.pallas_call(
        _dense_body,
        grid=grid,
        in_specs=[
            pl.BlockSpec((1, Qn, Dn), lambda b, j: (b, 0, 0)),
            pl.BlockSpec((1, _TL, Dn), lambda b, j: (b, j, 0)),
            pl.BlockSpec((1, 1, Ln), lambda b, j: (b, 0, 0)),
            pl.BlockSpec((1, 1, _TL), lambda b, j: (b, 0, j)),
            pl.BlockSpec((Rn, Dn), lambda b, j: (0, 0)),
            pl.BlockSpec((1, Rn), lambda b, j: (0, 0)),
            pl.BlockSpec((Rn, Dn), lambda b, j: (0, 0)),
            pl.BlockSpec((1, Rn), lambda b, j: (0, 0)),
            pl.BlockSpec((Hn, Dn), lambda b, j: (0, 0)),
            pl.BlockSpec((1, Hn), lambda b, j: (0, 0)),
            pl.BlockSpec((1, Hn), lambda b, j: (0, 0)),
            pl.BlockSpec(memory_space=pltpu.SMEM),
        ],
        out_specs=(
            pl.BlockSpec((1, 1, _TL), lambda b, j: (b, 0, j)),
            pl.BlockSpec((1, 1, _TL), lambda b, j: (b, 0, j)),
            pl.BlockSpec((1, 1, 128), lambda b, j: (b, 0, 0)),
        ),
        out_shape=out_shape,
        scratch_shapes=[pltpu.VMEM((1, Rn), jnp.float32)],
        compiler_params=pltpu.CompilerParams(
            dimension_semantics=("parallel", "arbitrary"),
        ),
    )(qe, ce, mask3, mask3, Wq, bq.reshape(1, Rn), Wc, bc.reshape(1, Rn),
      W1, b1.reshape(1, Hn), W2, b2.reshape(1, 1))
    return rel.reshape(Bn, Ln), keys.reshape(Bn, Ln), bud.reshape(Bn, 128)


def _select_body(keys_hbm, bud_hbm, out_hbm, keys_v, out_v, bud_v, L):
    nc = lax.axis_index("c")
    ns = lax.axis_index("s")
    wid = ns * 2 + nc
    pltpu.sync_copy(keys_hbm.at[wid], keys_v)
    pltpu.sync_copy(bud_hbm.at[wid], bud_v)
    k = lax.reduce_sum(bud_v[pl.ds(0, 16)], axes=(0,)) >> 4

    nchunk = L // 16
    group = 8

    def count_ge(thr):
        tv = jnp.full((16,), thr, jnp.uint32)

        def body(g, acc):
            for u in range(group):
                kv = keys_v[pl.ds((g * group + u) * 16, 16)]
                acc = acc + jnp.where(kv >= tv, 1, 0).astype(jnp.int32)
            return acc

        acc = lax.fori_loop(0, nchunk // group, body, jnp.zeros((16,), jnp.int32))
        return lax.reduce_sum(acc, axes=(0,))

    def bit_body(i, t):
        cand = t | (jnp.uint32(1) << (jnp.uint32(31) - i.astype(jnp.uint32)))
        c = count_ge(cand)
        return jnp.where(c >= k, cand, t)

    t = lax.fori_loop(0, 32, bit_body, jnp.uint32(0))
    tv = jnp.full((16,), t, jnp.uint32)

    def gt_body(g, acc):
        for u in range(group):
            kv = keys_v[pl.ds((g * group + u) * 16, 16)]
            acc = acc + jnp.where(kv > tv, 1, 0).astype(jnp.int32)
        return acc

    cnt_gt = lax.reduce_sum(
        lax.fori_loop(0, nchunk // group, gt_body, jnp.zeros((16,), jnp.int32)),
        axes=(0,))
    rem = k - cnt_gt

    def out_body(g, carry):
        for u in range(group):
            j = g * group + u
            kv = keys_v[pl.ds(j * 16, 16)]
            gt = kv > tv
            eq = kv == tv
            eqi = jnp.where(eq, 1, 0).astype(jnp.int32)
            pc = plsc.cumsum(eqi)
            sel = gt | (eq & ((carry + pc) <= rem))
            out_v[pl.ds(j * 16, 16)] = jnp.where(sel, 1.0, 0.0).astype(jnp.float32)
            carry = carry + lax.reduce_sum(eqi, axes=(0,))
        return carry

    lax.fori_loop(0, nchunk // group, out_body, jnp.int32(0))
    pltpu.sync_copy(out_v, out_hbm.at[wid])


def _select(keys, bud):
    Bn, Ln = keys.shape
    mesh = plsc.VectorSubcoreMesh(core_axis_name="c", subcore_axis_name="s")
    body = functools.partial(_select_body, L=Ln)
    return pl.kernel(
        body,
        mesh=mesh,
        out_type=jax.ShapeDtypeStruct((Bn, Ln), jnp.float32),
        scratch_types=[
            pltpu.VMEM((Ln,), jnp.uint32),
            pltpu.VMEM((Ln,), jnp.float32),
            pltpu.VMEM((128,), jnp.int32),
        ],
        compiler_params=pltpu.CompilerParams(needs_layout_passes=False),
    )(keys, bud)


def kernel(query_embeddings, context_embeddings, context_mask,
           Wq, bq, Wc, bc, W1, b1, W2, b2):
    rel, keys, bud = _dense(query_embeddings, context_embeddings, context_mask,
                            Wq, bq, Wc, bc, W1, b1, W2, b2)
    selection_mask = _select(keys, bud)
    return (selection_mask, rel)
